# baseline (speedup 1.0000x reference)
import functools

import jax
import jax.numpy as jnp
from jax import lax
from jax.experimental import pallas as pl
from jax.experimental.pallas import tpu as pltpu
from jax.experimental.pallas import tpu_sc as plsc

NUM_USERS = 100000
BATCH = 1024
HIST = 200

NC = 2
NS = 16
NW = NC * NS
ROWS_PER_W = BATCH // NW
WORDS_PER_ROW = 3 * HIST
L = 16
GROUPS = ROWS_PER_W // L


def _sc_body(in_hbm, w_hbm, pre_hbm, dist_hbm, theta_hbm,
             table_v, in_v, pre_v, dist_v, theta_v, sem_t, sem_i):
    wid = lax.axis_index("s") * NC + lax.axis_index("c")

    n_str = 4
    chunk = NUM_USERS // n_str
    cps = [pltpu.async_copy(w_hbm.at[pl.ds(k * chunk, chunk)],
                            table_v.at[pl.ds(k * chunk, chunk)], sem_t)
           for k in range(n_str)]
    cp_i = pltpu.async_copy(
        in_hbm.at[pl.ds(wid * ROWS_PER_W * WORDS_PER_ROW,
                        ROWS_PER_W * WORDS_PER_ROW)], in_v, sem_i)
    cp_i.wait()
    for cp in cps:
        cp.wait()

    iota = lax.iota(jnp.int32, L)
    zero = jnp.zeros((L,), jnp.float32)
    one = jnp.ones((L,), jnp.float32)

    for g in range(GROUPS):
        rowbase = (iota + g * L) * WORDS_PER_ROW

        def body(t, carry):
            valid, real, fake, cnt = carry
            idx = rowbase + 3 * t
            stance = plsc.load_gather(in_v, [idx])
            uid = plsc.load_gather(in_v, [idx + 2])
            uw = plsc.load_gather(table_v, [uid])
            valid = valid * jnp.where(stance != 3, one, zero)
            vm = uw * valid
            rc = jnp.where(stance == 0, vm, zero)
            real = real + rc
            fake = fake + (vm - rc)
            cnt = cnt + valid
            return valid, real, fake, cnt

        valid, real, fake, cnt = lax.fori_loop(
            0, HIST, body, (one, zero, zero, zero))

        m = jnp.maximum(real, fake)
        er = jnp.exp(real - m)
        ef = jnp.exp(fake - m)
        s = er + ef
        p0 = er / s
        p1 = ef / s
        th0 = p0 * cnt
        th1 = p1 * cnt
        ssum = th0 + th1
        mean = th1 / ssum
        var = th1 * th0 / (ssum * ssum * (ssum + 1.0))
        bits = plsc.bitcast(var, jnp.int32)
        y = plsc.bitcast(
            jnp.full((L,), 0x5F3759DF, jnp.int32)
            - lax.shift_right_logical(bits, jnp.ones((L,), jnp.int32)),
            jnp.float32)
        half_v = 0.5 * var
        for _ in range(3):
            y = y * (1.5 - half_v * y * y)
        std = var * y

        lo = 2 * iota + 2 * g * L
        hi = lo + 1
        plsc.store_scatter(pre_v, [lo], p0)
        plsc.store_scatter(pre_v, [hi], p1)
        plsc.store_scatter(dist_v, [lo], mean)
        plsc.store_scatter(dist_v, [hi], std)
        plsc.store_scatter(theta_v, [lo], th0)
        plsc.store_scatter(theta_v, [hi], th1)

    out_w = 2 * ROWS_PER_W
    pltpu.sync_copy(pre_v, pre_hbm.at[pl.ds(wid * out_w, out_w)])
    pltpu.sync_copy(dist_v, dist_hbm.at[pl.ds(wid * out_w, out_w)])
    pltpu.sync_copy(theta_v, theta_hbm.at[pl.ds(wid * out_w, out_w)])


@jax.jit
def kernel(inputs, w):
    flat_in = inputs.reshape(-1)
    out = jax.ShapeDtypeStruct((BATCH * 2,), jnp.float32)
    run = pl.kernel(
        _sc_body,
        out_type=(out, out, out),
        mesh=plsc.VectorSubcoreMesh(core_axis_name="c", subcore_axis_name="s"),
        scratch_types=[
            pltpu.VMEM((NUM_USERS,), jnp.float32),
            pltpu.VMEM((ROWS_PER_W * WORDS_PER_ROW,), jnp.int32),
            pltpu.VMEM((2 * ROWS_PER_W,), jnp.float32),
            pltpu.VMEM((2 * ROWS_PER_W,), jnp.float32),
            pltpu.VMEM((2 * ROWS_PER_W,), jnp.float32),
            pltpu.SemaphoreType.DMA,
            pltpu.SemaphoreType.DMA,
        ],
        compiler_params=pltpu.CompilerParams(needs_layout_passes=False),
    )
    pre, dist, theta = run(flat_in, w)
    return (pre.reshape(BATCH, 2), dist.reshape(BATCH, 2),
            theta.reshape(BATCH, 2))


# speedup vs baseline: 1.0467x; 1.0467x over previous
import functools

import jax
import jax.numpy as jnp
from jax import lax
from jax.experimental import pallas as pl
from jax.experimental.pallas import tpu as pltpu
from jax.experimental.pallas import tpu_sc as plsc

NUM_USERS = 100000
BATCH = 1024
HIST = 200

NC = 2
NS = 16
NW = NC * NS
ROWS_PER_W = BATCH // NW
WORDS_PER_ROW = 3 * HIST
L = 16
GROUPS = ROWS_PER_W // L


def _sc_body(in_hbm, w_hbm, pre_hbm, dist_hbm, theta_hbm,
             table_s, table_v, in_v, pre_v, dist_v, theta_v, sem_t, sem_i):
    cid = lax.axis_index("c")
    sid = lax.axis_index("s")
    wid = sid * NC + cid

    shard = 6256
    for k in range(NS):
        koff = min(k * shard, NUM_USERS - shard)

        @pl.when(sid == k)
        def _():
            pltpu.async_copy(w_hbm.at[pl.ds(koff, shard)],
                             table_v.at[pl.ds(0, shard)], sem_t).wait()
            pltpu.async_copy(table_v.at[pl.ds(0, shard)],
                             table_s.at[pl.ds(koff, shard)], sem_t).wait()
    cp_i = pltpu.async_copy(
        in_hbm.at[pl.ds(wid * ROWS_PER_W * WORDS_PER_ROW,
                        ROWS_PER_W * WORDS_PER_ROW)], in_v, sem_i)
    plsc.subcore_barrier()
    cp_t = pltpu.async_copy(table_s, table_v, sem_t)
    cp_i.wait()
    cp_t.wait()

    iota = lax.iota(jnp.int32, L)
    zero = jnp.zeros((L,), jnp.float32)
    one = jnp.ones((L,), jnp.float32)

    for g in range(GROUPS):
        rowbase = (iota + g * L) * WORDS_PER_ROW

        def body(t, carry):
            valid, real, fake, cnt = carry
            idx = rowbase + 3 * t
            stance = plsc.load_gather(in_v, [idx])
            uid = plsc.load_gather(in_v, [idx + 2])
            uw = plsc.load_gather(table_v, [uid])
            valid = valid * jnp.where(stance != 3, one, zero)
            vm = uw * valid
            rc = jnp.where(stance == 0, vm, zero)
            real = real + rc
            fake = fake + (vm - rc)
            cnt = cnt + valid
            return valid, real, fake, cnt

        valid, real, fake, cnt = lax.fori_loop(
            0, HIST, body, (one, zero, zero, zero))

        m = jnp.maximum(real, fake)
        er = jnp.exp(real - m)
        ef = jnp.exp(fake - m)
        s = er + ef
        p0 = er / s
        p1 = ef / s
        th0 = p0 * cnt
        th1 = p1 * cnt
        ssum = th0 + th1
        mean = th1 / ssum
        var = th1 * th0 / (ssum * ssum * (ssum + 1.0))
        bits = plsc.bitcast(var, jnp.int32)
        y = plsc.bitcast(
            jnp.full((L,), 0x5F3759DF, jnp.int32)
            - lax.shift_right_logical(bits, jnp.ones((L,), jnp.int32)),
            jnp.float32)
        half_v = 0.5 * var
        for _ in range(3):
            y = y * (1.5 - half_v * y * y)
        std = var * y

        lo = 2 * iota + 2 * g * L
        hi = lo + 1
        plsc.store_scatter(pre_v, [lo], p0)
        plsc.store_scatter(pre_v, [hi], p1)
        plsc.store_scatter(dist_v, [lo], mean)
        plsc.store_scatter(dist_v, [hi], std)
        plsc.store_scatter(theta_v, [lo], th0)
        plsc.store_scatter(theta_v, [hi], th1)

    out_w = 2 * ROWS_PER_W
    pltpu.sync_copy(pre_v, pre_hbm.at[pl.ds(wid * out_w, out_w)])
    pltpu.sync_copy(dist_v, dist_hbm.at[pl.ds(wid * out_w, out_w)])
    pltpu.sync_copy(theta_v, theta_hbm.at[pl.ds(wid * out_w, out_w)])


@jax.jit
def kernel(inputs, w):
    flat_in = inputs.reshape(-1)
    out = jax.ShapeDtypeStruct((BATCH * 2,), jnp.float32)
    run = pl.kernel(
        _sc_body,
        out_type=(out, out, out),
        mesh=plsc.VectorSubcoreMesh(core_axis_name="c", subcore_axis_name="s"),
        scratch_types=[
            pltpu.VMEM_SHARED((NUM_USERS,), jnp.float32),
            pltpu.VMEM((NUM_USERS,), jnp.float32),
            pltpu.VMEM((ROWS_PER_W * WORDS_PER_ROW,), jnp.int32),
            pltpu.VMEM((2 * ROWS_PER_W,), jnp.float32),
            pltpu.VMEM((2 * ROWS_PER_W,), jnp.float32),
            pltpu.VMEM((2 * ROWS_PER_W,), jnp.float32),
            pltpu.SemaphoreType.DMA,
            pltpu.SemaphoreType.DMA,
        ],
        compiler_params=pltpu.CompilerParams(needs_layout_passes=False),
    )
    pre, dist, theta = run(flat_in, w)
    return (pre.reshape(BATCH, 2), dist.reshape(BATCH, 2),
            theta.reshape(BATCH, 2))


# speedup vs baseline: 1.0574x; 1.0103x over previous
import functools

import jax
import jax.numpy as jnp
from jax import lax
from jax.experimental import pallas as pl
from jax.experimental.pallas import tpu as pltpu
from jax.experimental.pallas import tpu_sc as plsc

NUM_USERS = 100000
BATCH = 1024
HIST = 200

NC = 2
NS = 16
NW = NC * NS
ROWS_PER_W = BATCH // NW
WORDS_PER_ROW = 3 * HIST
L = 16
GROUPS = ROWS_PER_W // L


def _sc_body(in_hbm, w_hbm, pre_hbm, dist_hbm, theta_hbm,
             table_s, table_v, in_v, pre_v, dist_v, theta_v, sem_t, sem_i):
    cid = lax.axis_index("c")
    sid = lax.axis_index("s")
    wid = sid * NC + cid

    shard = 6256
    for k in range(NS):
        koff = min(k * shard, NUM_USERS - shard)

        @pl.when(sid == k)
        def _():
            pltpu.async_copy(w_hbm.at[pl.ds(koff, shard)],
                             table_v.at[pl.ds(0, shard)], sem_t).wait()
            pltpu.async_copy(table_v.at[pl.ds(0, shard)],
                             table_s.at[pl.ds(koff, shard)], sem_t).wait()
    cp_i = pltpu.async_copy(
        in_hbm.at[pl.ds(wid * ROWS_PER_W * WORDS_PER_ROW,
                        ROWS_PER_W * WORDS_PER_ROW)], in_v, sem_i)
    plsc.subcore_barrier()
    cp_t = pltpu.async_copy(table_s, table_v, sem_t)
    cp_i.wait()
    cp_t.wait()

    iota = lax.iota(jnp.int32, L)
    zero = jnp.zeros((L,), jnp.float32)
    one = jnp.ones((L,), jnp.float32)

    for g in range(GROUPS):
        rowbase = (iota + g * L) * WORDS_PER_ROW

        UNROLL = 4

        def body(i, carry):
            valid, real, fake, cnt = carry
            base = rowbase + (3 * UNROLL) * i
            stances = [plsc.load_gather(in_v, [base + 3 * u])
                       for u in range(UNROLL)]
            uids = [plsc.load_gather(in_v, [base + 3 * u + 2])
                    for u in range(UNROLL)]
            uws = [plsc.load_gather(table_v, [uid]) for uid in uids]
            for u in range(UNROLL):
                valid = valid * jnp.where(stances[u] != 3, one, zero)
                vm = uws[u] * valid
                rc = jnp.where(stances[u] == 0, vm, zero)
                real = real + rc
                fake = fake + (vm - rc)
                cnt = cnt + valid
            return valid, real, fake, cnt

        valid, real, fake, cnt = lax.fori_loop(
            0, HIST // UNROLL, body, (one, zero, zero, zero))

        m = jnp.maximum(real, fake)
        er = jnp.exp(real - m)
        ef = jnp.exp(fake - m)
        s = er + ef
        p0 = er / s
        p1 = ef / s
        th0 = p0 * cnt
        th1 = p1 * cnt
        ssum = th0 + th1
        mean = th1 / ssum
        var = th1 * th0 / (ssum * ssum * (ssum + 1.0))
        bits = plsc.bitcast(var, jnp.int32)
        y = plsc.bitcast(
            jnp.full((L,), 0x5F3759DF, jnp.int32)
            - lax.shift_right_logical(bits, jnp.ones((L,), jnp.int32)),
            jnp.float32)
        half_v = 0.5 * var
        for _ in range(3):
            y = y * (1.5 - half_v * y * y)
        std = var * y

        lo = 2 * iota + 2 * g * L
        hi = lo + 1
        plsc.store_scatter(pre_v, [lo], p0)
        plsc.store_scatter(pre_v, [hi], p1)
        plsc.store_scatter(dist_v, [lo], mean)
        plsc.store_scatter(dist_v, [hi], std)
        plsc.store_scatter(theta_v, [lo], th0)
        plsc.store_scatter(theta_v, [hi], th1)

    out_w = 2 * ROWS_PER_W
    pltpu.sync_copy(pre_v, pre_hbm.at[pl.ds(wid * out_w, out_w)])
    pltpu.sync_copy(dist_v, dist_hbm.at[pl.ds(wid * out_w, out_w)])
    pltpu.sync_copy(theta_v, theta_hbm.at[pl.ds(wid * out_w, out_w)])


@jax.jit
def kernel(inputs, w):
    flat_in = inputs.reshape(-1)
    out = jax.ShapeDtypeStruct((BATCH * 2,), jnp.float32)
    run = pl.kernel(
        _sc_body,
        out_type=(out, out, out),
        mesh=plsc.VectorSubcoreMesh(core_axis_name="c", subcore_axis_name="s"),
        scratch_types=[
            pltpu.VMEM_SHARED((NUM_USERS,), jnp.float32),
            pltpu.VMEM((NUM_USERS,), jnp.float32),
            pltpu.VMEM((ROWS_PER_W * WORDS_PER_ROW,), jnp.int32),
            pltpu.VMEM((2 * ROWS_PER_W,), jnp.float32),
            pltpu.VMEM((2 * ROWS_PER_W,), jnp.float32),
            pltpu.VMEM((2 * ROWS_PER_W,), jnp.float32),
            pltpu.SemaphoreType.DMA,
            pltpu.SemaphoreType.DMA,
        ],
        compiler_params=pltpu.CompilerParams(needs_layout_passes=False),
    )
    pre, dist, theta = run(flat_in, w)
    return (pre.reshape(BATCH, 2), dist.reshape(BATCH, 2),
            theta.reshape(BATCH, 2))
